# baseline (speedup 1.0000x reference)
import math

import jax
import jax.numpy as jnp
from jax.experimental import pallas as pl
from jax.experimental.pallas import tpu as pltpu

_E = 8
_K = 2
_C = 96
_B = 4
_H = 56
_W = 56
_HP = _H + 2
_WP = _W + 2
_P = _HP * _WP
_MARGIN = _WP + 1
_XE = ((_P + 2 * _MARGIN + 127) // 128) * 128
_NOISE_STD = 0.1
_OFFS = tuple((dy - 1) * _WP + (dx - 1) for dy in range(3) for dx in range(3))


def _gelu(x):
    return 0.5 * x * (1.0 + jax.lax.erf(x * (1.0 / math.sqrt(2.0))))


def _moe_body(tf_ref, wg_ref, noise_ref, xp_ref, mask_ref, w1_ref, b1_ref,
              w2_ref, b2_ref, out_ref, xext, xcat, hext, hacat, hbcat):
    b = pl.program_id(0)

    t = tf_ref[0]
    wg = wg_ref[...]
    logits = jax.lax.dot_general(
        t, wg, (((1,), (1,)), ((), ())),
        preferred_element_type=jnp.float32)
    logits = logits + noise_ref[0]
    m = jnp.max(logits, axis=-1, keepdims=True)
    e = jnp.exp(logits - m)
    w = e / jnp.sum(e, axis=-1, keepdims=True)
    col = jax.lax.broadcasted_iota(jnp.int32, w.shape, 1)
    v0 = jnp.max(w)
    i0 = jnp.min(jnp.where(w == v0, col, _E))
    w2 = jnp.where(col == i0, -1.0, w)
    v1 = jnp.max(w2)
    i1 = jnp.min(jnp.where(w2 == v1, col, _E))
    s0 = v0
    s1 = v1

    @pl.when(b == 0)
    def _init():
        xext[...] = jnp.zeros((_C, _XE), jnp.bfloat16)
        hext[...] = jnp.zeros((2 * _C, _XE), jnp.bfloat16)

    xext[:, _MARGIN:_MARGIN + _P] = xp_ref[0]
    for t_, o in enumerate(_OFFS):
        xcat[t_ * _C:(t_ + 1) * _C, :] = xext[:, _MARGIN + o:_MARGIN + o + _P]

    w1a = w1_ref[i0]
    w1b = w1_ref[i1]
    y1a = jax.lax.dot_general(
        w1a, xcat[...], (((1,), (0,)), ((), ())),
        preferred_element_type=jnp.float32)
    y1b = jax.lax.dot_general(
        w1b, xcat[...], (((1,), (0,)), ((), ())),
        preferred_element_type=jnp.float32)
    bias1 = jnp.concatenate([b1_ref[i0], b1_ref[i1]], axis=0)
    h = _gelu(jnp.concatenate([y1a, y1b], axis=0) + bias1)

    rows = jax.lax.broadcasted_iota(jnp.int32, (2 * _C, 1), 0)
    scale = jnp.where(rows < _C, s0, s1)
    h = h * mask_ref[0] * scale

    hext[:, _MARGIN:_MARGIN + _P] = h.astype(jnp.bfloat16)
    for t_, o in enumerate(_OFFS):
        hacat[t_ * _C:(t_ + 1) * _C, :] = hext[0:_C, _MARGIN + o:_MARGIN + o + _P]
        hbcat[t_ * _C:(t_ + 1) * _C, :] = hext[_C:2 * _C, _MARGIN + o:_MARGIN + o + _P]

    y2 = jax.lax.dot_general(
        w2_ref[i0], hacat[...], (((1,), (0,)), ((), ())),
        preferred_element_type=jnp.float32)
    y2 = y2 + jax.lax.dot_general(
        w2_ref[i1], hbcat[...], (((1,), (0,)), ((), ())),
        preferred_element_type=jnp.float32)
    bias2 = s0 * b2_ref[i0] + s1 * b2_ref[i1]
    out_ref[0] = y2 + bias2


@jax.jit
def kernel(x, text_feature, training, Wg, W1, b1, W2, b2):
    B = x.shape[0]
    noise = jax.random.normal(jax.random.key(42), (B, _E), jnp.float32) * _NOISE_STD
    noise_eff = jnp.where(jnp.asarray(training) != 0, noise, 0.0)

    xp = jnp.pad(x, ((0, 0), (0, 0), (1, 1), (1, 1)))
    xp = xp.reshape(B, _C, _P).astype(jnp.bfloat16)
    ii = jnp.arange(_P, dtype=jnp.int32) // _WP
    jj = jnp.arange(_P, dtype=jnp.int32) % _WP
    mask = ((ii >= 1) & (ii <= _H) & (jj >= 1) & (jj <= _W))
    mask = mask.astype(jnp.float32).reshape(1, 1, _P)
    W1t = W1.transpose(0, 1, 3, 4, 2).reshape(_E, _C, 9 * _C).astype(jnp.bfloat16)
    W2t = W2.transpose(0, 1, 3, 4, 2).reshape(_E, _C, 9 * _C).astype(jnp.bfloat16)
    b1r = b1.reshape(_E, _C, 1)
    b2r = b2.reshape(_E, _C, 1)

    out = pl.pallas_call(
        _moe_body,
        grid=(B,),
        in_specs=[
            pl.BlockSpec((1, 1, 512), lambda b: (b, 0, 0)),
            pl.BlockSpec((_E, 512), lambda b: (0, 0)),
            pl.BlockSpec((1, 1, _E), lambda b: (b, 0, 0)),
            pl.BlockSpec((1, _C, _P), lambda b: (b, 0, 0)),
            pl.BlockSpec((1, 1, _P), lambda b: (0, 0, 0)),
            pl.BlockSpec((_E, _C, 9 * _C), lambda b: (0, 0, 0)),
            pl.BlockSpec((_E, _C, 1), lambda b: (0, 0, 0)),
            pl.BlockSpec((_E, _C, 9 * _C), lambda b: (0, 0, 0)),
            pl.BlockSpec((_E, _C, 1), lambda b: (0, 0, 0)),
        ],
        out_specs=pl.BlockSpec((1, _C, _P), lambda b: (b, 0, 0)),
        scratch_shapes=[
            pltpu.VMEM((_C, _XE), jnp.bfloat16),
            pltpu.VMEM((9 * _C, _P), jnp.bfloat16),
            pltpu.VMEM((2 * _C, _XE), jnp.bfloat16),
            pltpu.VMEM((9 * _C, _P), jnp.bfloat16),
            pltpu.VMEM((9 * _C, _P), jnp.bfloat16),
        ],
        out_shape=jax.ShapeDtypeStruct((B, _C, _P), jnp.float32),
        compiler_params=pltpu.CompilerParams(
            dimension_semantics=("arbitrary",)),
    )(text_feature.reshape(B, 1, 512), Wg, noise_eff.reshape(B, 1, _E),
      xp, mask, W1t, b1r, W2t, b2r)

    return out.reshape(B, _C, _HP, _WP)[:, :, 1:-1, 1:-1]
